# baseline (speedup 1.0000x reference)
import functools

import jax
import jax.numpy as jnp
from jax import lax
from jax.experimental import pallas as pl
from jax.experimental.pallas import tpu as pltpu
from jax.experimental.pallas import tpu_sc as plsc

_N = 4096
_D = _N + 1
_EPS = 1e-6
_K = 16
_RPW = 256
_NCHUNK = _RPW // _K


def _spu(x):
    return jnp.where(x >= 0, x * x - 0.5, jax.nn.sigmoid(-x) - 1.0)


def _spu_grad(x):
    s = jax.nn.sigmoid(-x)
    return jnp.where(x >= 0, 2.0 * x, -s * (1.0 - s))


def _diff_clamp(x, a, b):
    return jnp.tanh(x) * (b - a) / 2.0 + (b + a) / 2.0


def _params_body(lb_ref, ub_ref, sl_ref, su_ref,
                 slu_ref, suu_ref, lbias_ref, ubias_ref):
    lb = lb_ref[...]
    ub = ub_ref[...]
    slope_l = sl_ref[...]
    slope_u = su_ref[...]
    spu_ub = _spu(ub)
    spu_lb = _spu(lb)
    g_ub = _spu_grad(ub)
    g_lb = _spu_grad(lb)
    mask_1 = lb >= 0
    mask_2 = ub <= 0
    a = (spu_ub - spu_lb) / (ub - lb + _EPS)
    zeros = jnp.zeros_like(a)
    slope_u_use = jnp.where(
        mask_1,
        _diff_clamp(slope_u, a, a),
        jnp.where(
            mask_2,
            _diff_clamp(slope_u, g_ub, g_lb),
            _diff_clamp(slope_u, jnp.full_like(a, -0.25), jnp.maximum(zeros, a)),
        ),
    )
    slope_l_use = jnp.where(
        mask_1,
        _diff_clamp(slope_l, g_lb, g_ub),
        jnp.where(
            mask_2,
            _diff_clamp(slope_l, a, a),
            _diff_clamp(slope_l, (spu_lb + 0.5) / (lb + _EPS), g_ub),
        ),
    )
    b1 = spu_lb - slope_l_use * lb
    b2 = spu_ub - slope_l_use * ub
    l_bias = jnp.minimum(b1, b2)
    c1 = spu_lb - slope_u_use * lb
    c2 = spu_ub - slope_u_use * ub
    xv = slope_u_use / 2.0
    valid = (xv >= jnp.maximum(lb, 0.0)) & (xv <= ub)
    c3 = jnp.where(valid, -slope_u_use * slope_u_use / 4.0 - 0.5, -1e30)
    u_bias = jnp.maximum(jnp.maximum(c1, c2), c3)
    slu_ref[...] = slope_l_use
    suu_ref[...] = slope_u_use
    lbias_ref[...] = l_bias
    ubias_ref[...] = u_bias


def _compute_params(lb, ub, slope_l, slope_u):
    shape2d = (32, 128)
    args = [x.reshape(shape2d) for x in (lb, ub, slope_l, slope_u)]
    o = jax.ShapeDtypeStruct(shape2d, jnp.float32)
    slu, suu, lbias, ubias = pl.pallas_call(
        _params_body,
        out_shape=[o, o, o, o],
    )(*args)
    return (slu.reshape(_N), suu.reshape(_N),
            lbias.reshape(_N), ubias.reshape(_N))


def _sc_fill_body(slu_hbm, lbias_hbm, suu_hbm, ubias_hbm, zrows_hbm,
                  out_hbm, slope_v, bias_v, buf, sem):
    c = lax.axis_index("c")
    s = lax.axis_index("s")

    @pl.when(c == 0)
    def _():
        pltpu.sync_copy(slu_hbm, slope_v)
        pltpu.sync_copy(lbias_hbm, bias_v)

    @pl.when(c != 0)
    def _():
        pltpu.sync_copy(suu_hbm, slope_v)
        pltpu.sync_copy(ubias_hbm, bias_v)

    pltpu.sync_copy(zrows_hbm, buf)

    lane = lax.iota(jnp.int32, 16)
    lastcol = jnp.full((16,), _N, jnp.int32)
    zval = jnp.zeros((16,), jnp.float32)
    row0 = s * _RPW

    def chunk(i, carry):
        r0 = row0 + i * _K
        diag_col = r0 + lane
        slope16 = slope_v[pl.ds(r0, _K)]
        bias16 = bias_v[pl.ds(r0, _K)]
        plsc.store_scatter(buf, [lane, diag_col], slope16)
        plsc.store_scatter(buf, [lane, lastcol], bias16)
        pltpu.async_copy(buf, out_hbm.at[c, pl.ds(r0, _K), :], sem).wait()
        plsc.store_scatter(buf, [lane, diag_col], zval)
        plsc.store_scatter(buf, [lane, lastcol], zval)
        return carry

    lax.fori_loop(0, _NCHUNK, chunk, 0)

    @pl.when(s == 15)
    def _():
        one_idx = jnp.full((16,), _N, jnp.int32)
        zero_row = jnp.zeros((16,), jnp.int32)
        one_val = jnp.full((16,), 1.0, jnp.float32)
        plsc.store_scatter(buf, [zero_row, one_idx], one_val)
        pltpu.async_copy(buf.at[pl.ds(0, 1), :],
                         out_hbm.at[c, pl.ds(_N, 1), :], sem).wait()


def _sc_fill(slu, suu, lbias, ubias):
    zrows = jnp.zeros((_K, _D), jnp.float32)
    mesh = plsc.VectorSubcoreMesh(core_axis_name="c", subcore_axis_name="s")
    fill = functools.partial(
        pl.kernel,
        mesh=mesh,
        compiler_params=pltpu.CompilerParams(needs_layout_passes=False),
        out_type=jax.ShapeDtypeStruct((2, _D, _D), jnp.float32),
        scratch_types=[
            pltpu.VMEM((_N,), jnp.float32),
            pltpu.VMEM((_N,), jnp.float32),
            pltpu.VMEM((_K, _D), jnp.float32),
            pltpu.SemaphoreType.DMA,
        ],
    )(_sc_fill_body)
    return fill(slu, lbias, suu, ubias, zrows)


def kernel(lb, ub, slope_l, slope_u):
    slu, suu, lbias, ubias = _compute_params(lb, ub, slope_l, slope_u)
    return _sc_fill(slu, suu, lbias, ubias)


# speedup vs baseline: 5.3530x; 5.3530x over previous
import functools

import jax
import jax.numpy as jnp
from jax import lax
from jax.experimental import pallas as pl
from jax.experimental.pallas import tpu as pltpu
from jax.experimental.pallas import tpu_sc as plsc

_N = 4096
_D = _N + 1
_EPS = 1e-6
_K = 8
_RPW = 128
_NCHUNK = _RPW // _K
_VPAD = _N + 16


def _spu(x):
    return jnp.where(x >= 0, x * x - 0.5, jax.nn.sigmoid(-x) - 1.0)


def _spu_grad(x):
    s = jax.nn.sigmoid(-x)
    return jnp.where(x >= 0, 2.0 * x, -s * (1.0 - s))


def _diff_clamp(x, a, b):
    return jnp.tanh(x) * (b - a) / 2.0 + (b + a) / 2.0


def _params_body(lb_ref, ub_ref, sl_ref, su_ref,
                 slu_ref, suu_ref, lbias_ref, ubias_ref):
    lb = lb_ref[...]
    ub = ub_ref[...]
    slope_l = sl_ref[...]
    slope_u = su_ref[...]
    spu_ub = _spu(ub)
    spu_lb = _spu(lb)
    g_ub = _spu_grad(ub)
    g_lb = _spu_grad(lb)
    mask_1 = lb >= 0
    mask_2 = ub <= 0
    a = (spu_ub - spu_lb) / (ub - lb + _EPS)
    zeros = jnp.zeros_like(a)
    slope_u_use = jnp.where(
        mask_1,
        _diff_clamp(slope_u, a, a),
        jnp.where(
            mask_2,
            _diff_clamp(slope_u, g_ub, g_lb),
            _diff_clamp(slope_u, jnp.full_like(a, -0.25), jnp.maximum(zeros, a)),
        ),
    )
    slope_l_use = jnp.where(
        mask_1,
        _diff_clamp(slope_l, g_lb, g_ub),
        jnp.where(
            mask_2,
            _diff_clamp(slope_l, a, a),
            _diff_clamp(slope_l, (spu_lb + 0.5) / (lb + _EPS), g_ub),
        ),
    )
    b1 = spu_lb - slope_l_use * lb
    b2 = spu_ub - slope_l_use * ub
    l_bias = jnp.minimum(b1, b2)
    c1 = spu_lb - slope_u_use * lb
    c2 = spu_ub - slope_u_use * ub
    xv = slope_u_use / 2.0
    valid = (xv >= jnp.maximum(lb, 0.0)) & (xv <= ub)
    c3 = jnp.where(valid, -slope_u_use * slope_u_use / 4.0 - 0.5, -1e30)
    u_bias = jnp.maximum(jnp.maximum(c1, c2), c3)
    slu_ref[...] = slope_l_use
    suu_ref[...] = slope_u_use
    lbias_ref[...] = l_bias
    ubias_ref[...] = u_bias


def _compute_params(lb, ub, slope_l, slope_u):
    shape2d = (32, 128)
    args = [x.reshape(shape2d) for x in (lb, ub, slope_l, slope_u)]
    o = jax.ShapeDtypeStruct(shape2d, jnp.float32)
    slu, suu, lbias, ubias = pl.pallas_call(
        _params_body,
        out_shape=[o, o, o, o],
    )(*args)
    return (slu.reshape(_N), suu.reshape(_N),
            lbias.reshape(_N), ubias.reshape(_N))


def _sc_fill_body(slu_hbm, suu_hbm, lbias_hbm, ubias_hbm, zrows_hbm,
                  out_hbm, slu_v, suu_v, lb_v, ub_v, buf, sem):
    c = lax.axis_index("c")
    s = lax.axis_index("s")
    w = c * 16 + s

    pltpu.sync_copy(slu_hbm, slu_v.at[pl.ds(0, _N)])
    pltpu.sync_copy(suu_hbm, suu_v.at[pl.ds(0, _N)])
    pltpu.sync_copy(lbias_hbm, lb_v.at[pl.ds(0, _N)])
    pltpu.sync_copy(ubias_hbm, ub_v.at[pl.ds(0, _N)])

    pltpu.sync_copy(zrows_hbm, buf)

    lane = lax.iota(jnp.int32, 16)
    rr = lane & 7
    mlow = lane < 8
    zero16i = jnp.zeros((16,), jnp.int32)
    one16i = jnp.full((16,), 1, jnp.int32)
    last16i = jnp.full((16,), _N, jnp.int32)
    zval = jnp.zeros((16,), jnp.float32)
    row0 = w * _RPW

    def chunk(i, carry):
        r0 = row0 + i * _K
        dcol = r0 + rr
        slu16 = slu_v[pl.ds(r0, 16)]
        suu16 = suu_v[pl.ds(r0, 16)]
        lb16 = lb_v[pl.ds(r0, 16)]
        ub16 = ub_v[pl.ds(r0, 16)]
        plsc.store_scatter(buf, [rr, zero16i, dcol], slu16, mask=mlow)
        plsc.store_scatter(buf, [rr, one16i, dcol], suu16, mask=mlow)
        plsc.store_scatter(buf, [rr, zero16i, last16i], lb16, mask=mlow)
        plsc.store_scatter(buf, [rr, one16i, last16i], ub16, mask=mlow)
        pltpu.async_copy(buf, out_hbm.at[pl.ds(r0, _K)], sem).wait()
        plsc.store_scatter(buf, [rr, zero16i, dcol], zval, mask=mlow)
        plsc.store_scatter(buf, [rr, one16i, dcol], zval, mask=mlow)
        plsc.store_scatter(buf, [rr, zero16i, last16i], zval, mask=mlow)
        plsc.store_scatter(buf, [rr, one16i, last16i], zval, mask=mlow)
        return carry

    lax.fori_loop(0, _NCHUNK, chunk, 0)

    @pl.when(w == 31)
    def _():
        m_idx = jnp.where(lane < 8, 0, 1).astype(jnp.int32)
        one_val = jnp.full((16,), 1.0, jnp.float32)
        plsc.store_scatter(buf, [zero16i, m_idx, last16i], one_val)
        pltpu.async_copy(buf.at[pl.ds(0, 1)],
                         out_hbm.at[pl.ds(_N, 1)], sem).wait()


def _sc_fill(slu, suu, lbias, ubias):
    zrows = jnp.zeros((_K, 2, _D), jnp.float32)
    mesh = plsc.VectorSubcoreMesh(core_axis_name="c", subcore_axis_name="s")
    fill = functools.partial(
        pl.kernel,
        mesh=mesh,
        compiler_params=pltpu.CompilerParams(needs_layout_passes=False),
        out_type=jax.ShapeDtypeStruct((_D, 2, _D), jnp.float32),
        scratch_types=[
            pltpu.VMEM((_VPAD,), jnp.float32),
            pltpu.VMEM((_VPAD,), jnp.float32),
            pltpu.VMEM((_VPAD,), jnp.float32),
            pltpu.VMEM((_VPAD,), jnp.float32),
            pltpu.VMEM((_K, 2, _D), jnp.float32),
            pltpu.SemaphoreType.DMA,
        ],
    )(_sc_fill_body)
    return fill(slu, suu, lbias, ubias, zrows)


def kernel(lb, ub, slope_l, slope_u):
    slu, suu, lbias, ubias = _compute_params(lb, ub, slope_l, slope_u)
    out = _sc_fill(slu, suu, lbias, ubias)
    return jnp.transpose(out, (1, 0, 2))
